# baseline (speedup 1.0000x reference)
import jax
import jax.numpy as jnp
from jax.experimental import pallas as pl
from jax.experimental.pallas import tpu as pltpu


def _copy3_body(x_ref, a_ref, f_ref, ao_ref, fo_ref, xo_ref,
                sem_a, sem_f, sem_x):
    cp_a = pltpu.make_async_copy(a_ref, ao_ref, sem_a)
    cp_f = pltpu.make_async_copy(f_ref, fo_ref, sem_f)
    cp_x = pltpu.make_async_copy(x_ref, xo_ref, sem_x)
    cp_a.start()
    cp_f.start()
    cp_x.start()
    cp_a.wait()
    cp_f.wait()
    cp_x.wait()


def kernel(x, author_embed, field_embed):
    out = pl.pallas_call(
        _copy3_body,
        in_specs=[
            pl.BlockSpec(memory_space=pl.ANY),
            pl.BlockSpec(memory_space=pl.ANY),
            pl.BlockSpec(memory_space=pl.ANY),
        ],
        out_specs=[
            pl.BlockSpec(memory_space=pl.ANY),
            pl.BlockSpec(memory_space=pl.ANY),
            pl.BlockSpec(memory_space=pl.ANY),
        ],
        out_shape=[
            jax.ShapeDtypeStruct(author_embed.shape, author_embed.dtype),
            jax.ShapeDtypeStruct(field_embed.shape, field_embed.dtype),
            jax.ShapeDtypeStruct(x.shape, x.dtype),
        ],
        scratch_shapes=[
            pltpu.SemaphoreType.DMA,
            pltpu.SemaphoreType.DMA,
            pltpu.SemaphoreType.DMA,
        ],
    )(x, author_embed, field_embed)
    return (out[0], out[1], out[2])


# speedup vs baseline: 1.0008x; 1.0008x over previous
import jax
import jax.numpy as jnp
from jax.experimental import pallas as pl
from jax.experimental.pallas import tpu as pltpu

_CHUNKS_BIG = 32
_ROWS_BIG = 100000 // _CHUNKS_BIG
_CHUNKS_X = 16
_ROWS_X = 50000 // _CHUNKS_X


def _copy3_body(x_ref, a_ref, f_ref, ao_ref, fo_ref, xo_ref, sem):
    for i in range(_CHUNKS_BIG):
        sl = pl.ds(i * _ROWS_BIG, _ROWS_BIG)
        pltpu.make_async_copy(a_ref.at[sl], ao_ref.at[sl], sem).start()
        pltpu.make_async_copy(f_ref.at[sl], fo_ref.at[sl], sem).start()
    for i in range(_CHUNKS_X):
        sl = pl.ds(i * _ROWS_X, _ROWS_X)
        pltpu.make_async_copy(x_ref.at[sl], xo_ref.at[sl], sem).start()
    for i in range(_CHUNKS_BIG):
        sl = pl.ds(i * _ROWS_BIG, _ROWS_BIG)
        pltpu.make_async_copy(a_ref.at[sl], ao_ref.at[sl], sem).wait()
        pltpu.make_async_copy(f_ref.at[sl], fo_ref.at[sl], sem).wait()
    for i in range(_CHUNKS_X):
        sl = pl.ds(i * _ROWS_X, _ROWS_X)
        pltpu.make_async_copy(x_ref.at[sl], xo_ref.at[sl], sem).wait()


def kernel(x, author_embed, field_embed):
    out = pl.pallas_call(
        _copy3_body,
        in_specs=[
            pl.BlockSpec(memory_space=pl.ANY),
            pl.BlockSpec(memory_space=pl.ANY),
            pl.BlockSpec(memory_space=pl.ANY),
        ],
        out_specs=[
            pl.BlockSpec(memory_space=pl.ANY),
            pl.BlockSpec(memory_space=pl.ANY),
            pl.BlockSpec(memory_space=pl.ANY),
        ],
        out_shape=[
            jax.ShapeDtypeStruct(author_embed.shape, author_embed.dtype),
            jax.ShapeDtypeStruct(field_embed.shape, field_embed.dtype),
            jax.ShapeDtypeStruct(x.shape, x.dtype),
        ],
        scratch_shapes=[pltpu.SemaphoreType.DMA],
    )(x, author_embed, field_embed)
    return (out[0], out[1], out[2])


# speedup vs baseline: 47.7369x; 47.6968x over previous
import jax
import jax.numpy as jnp
from jax.experimental import pallas as pl
from jax.experimental.pallas import tpu as pltpu

_GRID = 25
_ROWS_BIG = 100000 // _GRID
_ROWS_X = 50000 // _GRID
_D = 128


def _copy3_body(x_ref, a_ref, f_ref, ao_ref, fo_ref, xo_ref):
    ao_ref[...] = a_ref[...]
    fo_ref[...] = f_ref[...]
    xo_ref[...] = x_ref[...]


def kernel(x, author_embed, field_embed):
    out = pl.pallas_call(
        _copy3_body,
        grid=(_GRID,),
        in_specs=[
            pl.BlockSpec((_ROWS_X, _D), lambda i: (i, 0)),
            pl.BlockSpec((_ROWS_BIG, _D), lambda i: (i, 0)),
            pl.BlockSpec((_ROWS_BIG, _D), lambda i: (i, 0)),
        ],
        out_specs=[
            pl.BlockSpec((_ROWS_BIG, _D), lambda i: (i, 0)),
            pl.BlockSpec((_ROWS_BIG, _D), lambda i: (i, 0)),
            pl.BlockSpec((_ROWS_X, _D), lambda i: (i, 0)),
        ],
        out_shape=[
            jax.ShapeDtypeStruct(author_embed.shape, author_embed.dtype),
            jax.ShapeDtypeStruct(field_embed.shape, field_embed.dtype),
            jax.ShapeDtypeStruct(x.shape, x.dtype),
        ],
        compiler_params=pltpu.CompilerParams(
            dimension_semantics=("parallel",),
        ),
    )(x, author_embed, field_embed)
    return (out[0], out[1], out[2])


# speedup vs baseline: 48.9157x; 1.0247x over previous
import jax
import jax.numpy as jnp
from jax.experimental import pallas as pl
from jax.experimental.pallas import tpu as pltpu

_GRID = 10
_ROWS_BIG = 100000 // _GRID
_ROWS_X = 50000 // _GRID
_D = 128


def _copy3_body(x_ref, a_ref, f_ref, ao_ref, fo_ref, xo_ref):
    ao_ref[...] = a_ref[...]
    fo_ref[...] = f_ref[...]
    xo_ref[...] = x_ref[...]


def kernel(x, author_embed, field_embed):
    out = pl.pallas_call(
        _copy3_body,
        grid=(_GRID,),
        in_specs=[
            pl.BlockSpec((_ROWS_X, _D), lambda i: (i, 0)),
            pl.BlockSpec((_ROWS_BIG, _D), lambda i: (i, 0)),
            pl.BlockSpec((_ROWS_BIG, _D), lambda i: (i, 0)),
        ],
        out_specs=[
            pl.BlockSpec((_ROWS_BIG, _D), lambda i: (i, 0)),
            pl.BlockSpec((_ROWS_BIG, _D), lambda i: (i, 0)),
            pl.BlockSpec((_ROWS_X, _D), lambda i: (i, 0)),
        ],
        out_shape=[
            jax.ShapeDtypeStruct(author_embed.shape, author_embed.dtype),
            jax.ShapeDtypeStruct(field_embed.shape, field_embed.dtype),
            jax.ShapeDtypeStruct(x.shape, x.dtype),
        ],
        compiler_params=pltpu.CompilerParams(
            dimension_semantics=("parallel",),
        ),
    )(x, author_embed, field_embed)
    return (out[0], out[1], out[2])
